# Initial kernel scaffold; baseline (speedup 1.0000x reference)
import functools

import jax
import jax.numpy as jnp
from jax import lax
from jax.experimental import pallas as pl
from jax.experimental.pallas import tpu as pltpu
from jax.experimental.pallas import tpu_sc as plsc

DS_SIZE = 100000
DIM = 64
BATCH = 1024

_info = plsc.get_sparse_core_info()
_NC, _NS = _info.num_cores, _info.num_subcores
_NW = _NC * _NS
_BPW = BATCH // _NW


def _gather_body(mem_hbm, idx_hbm, out_hbm, idx_v, rows_v, sem):
    wid = lax.axis_index("s") * _NC + lax.axis_index("c")
    base = wid * _BPW
    pltpu.sync_copy(idx_hbm.at[pl.ds(base, _BPW)], idx_v)
    pltpu.async_copy(mem_hbm.at[idx_v], rows_v, sem).wait()
    pltpu.sync_copy(rows_v, out_hbm.at[pl.ds(base, _BPW)])


_gather = functools.partial(
    pl.kernel,
    mesh=plsc.VectorSubcoreMesh(core_axis_name="c", subcore_axis_name="s"),
    out_type=jax.ShapeDtypeStruct((BATCH, DIM), jnp.float32),
    scratch_types=[
        pltpu.VMEM((_BPW,), jnp.int32),
        pltpu.VMEM((_BPW, DIM), jnp.float32),
        pltpu.SemaphoreType.DMA,
    ],
)(_gather_body)


def kernel(mem, target, idx):
    out = _gather(mem, idx.astype(jnp.int32))
    return (out, target)


# speedup vs baseline: 7.9434x; 7.9434x over previous
import functools

import jax
import jax.numpy as jnp
from jax import lax
from jax.experimental import pallas as pl
from jax.experimental.pallas import tpu as pltpu
from jax.experimental.pallas import tpu_sc as plsc

DS_SIZE = 100000
DIM = 64
BATCH = 1024

_info = plsc.get_sparse_core_info()
_NC, _NS = _info.num_cores, _info.num_subcores
_NW = _NC * _NS
_BPW = BATCH // _NW


def _gather_body(mem_hbm, idx_hbm, out_hbm, idx_v, rows_v, sem):
    wid = lax.axis_index("s") * _NC + lax.axis_index("c")
    base = wid * _BPW
    pltpu.sync_copy(idx_hbm.at[pl.ds(base, _BPW)], idx_v)
    pltpu.async_copy(mem_hbm.at[idx_v], rows_v, sem).wait()
    pltpu.sync_copy(rows_v, out_hbm.at[pl.ds(base, _BPW)])


_gather = functools.partial(
    pl.kernel,
    mesh=plsc.VectorSubcoreMesh(core_axis_name="c", subcore_axis_name="s"),
    out_type=jax.ShapeDtypeStruct((BATCH, DIM), jnp.float32),
    scratch_types=[
        pltpu.VMEM((_BPW,), jnp.int32),
        pltpu.VMEM((_BPW, DIM), jnp.float32),
        pltpu.SemaphoreType.DMA,
    ],
    compiler_params=pltpu.CompilerParams(use_tc_tiling_on_sc=False),
)(_gather_body)


def kernel(mem, target, idx):
    out = _gather(mem, idx.astype(jnp.int32))
    return (out, target)


# speedup vs baseline: 11.7216x; 1.4756x over previous
import functools

import jax
import jax.numpy as jnp
from jax import lax
from jax.experimental import pallas as pl
from jax.experimental.pallas import tpu as pltpu
from jax.experimental.pallas import tpu_sc as plsc

DS_SIZE = 100000
DIM = 64
BATCH = 1024

_info = plsc.get_sparse_core_info()
_NC, _NS, _L = _info.num_cores, _info.num_subcores, _info.num_lanes
_NW = _NC * _NS
_BPW = BATCH // _NW


def _gather_body(mem_hbm, idx_hbm, out_hbm, idx_v, out_v, sem):
    wid = lax.axis_index("s") * _NC + lax.axis_index("c")
    base = wid * _BPW
    pltpu.sync_copy(idx_hbm.at[pl.ds(base, _BPW)], idx_v)
    lane = lax.iota(jnp.int32, _L)
    copies = []
    for h in range(_BPW // _L):
        v = idx_v[pl.ds(h * _L, _L)]
        for j in range(_L):
            row = jnp.sum(jnp.where(lane == j, v, 0))
            b = h * _L + j
            copies.append(pltpu.async_copy(mem_hbm.at[row], out_v.at[b], sem))
    for cp in copies:
        cp.wait()
    pltpu.sync_copy(out_v, out_hbm.at[pl.ds(base, _BPW)])


_gather = functools.partial(
    pl.kernel,
    mesh=plsc.VectorSubcoreMesh(core_axis_name="c", subcore_axis_name="s"),
    out_type=jax.ShapeDtypeStruct((BATCH, DIM), jnp.float32),
    scratch_types=[
        pltpu.VMEM((_BPW,), jnp.int32),
        pltpu.VMEM((_BPW, DIM), jnp.float32),
        pltpu.SemaphoreType.DMA,
    ],
    compiler_params=pltpu.CompilerParams(needs_layout_passes=False),
)(_gather_body)


def kernel(mem, target, idx):
    out = _gather(mem, idx.astype(jnp.int32))
    return (out, target)


# speedup vs baseline: 15.2964x; 1.3050x over previous
import functools

import jax
import jax.numpy as jnp
from jax import lax
from jax.experimental import pallas as pl
from jax.experimental.pallas import tpu as pltpu
from jax.experimental.pallas import tpu_sc as plsc

DS_SIZE = 100000
DIM = 64
BATCH = 1024

_info = plsc.get_sparse_core_info()
_NC, _NS, _L = _info.num_cores, _info.num_subcores, _info.num_lanes
_NGRP = 8
_GF = DIM // _NGRP
_CPT = 4
_W = 6272
_PAD_COLS = 100096
_TAIL_OFF = _PAD_COLS - _W


def _gather_body(memt_hbm, idx_hbm, outt_hbm, idx_v, buf_a, buf_b, outp_v,
                 tmp_v, shared, sem_a, sem_b):
    c = lax.axis_index("c")
    s = lax.axis_index("s")
    grp = c * 4 + s // 4
    cpos = s % 4
    f0 = pl.multiple_of(grp * _GF, _GF)

    pltpu.sync_copy(idx_hbm, idx_v)

    bufs = (buf_a, buf_b)
    sems = (sem_a, sem_b)

    def chunk_off(j):
        off = jnp.minimum((cpos * _CPT + j) * _W, _TAIL_OFF)
        return pl.multiple_of(off, 128)

    def fire(j):
        return pltpu.async_copy(
            memt_hbm.at[pl.ds(f0, _GF), pl.ds(chunk_off(j), _W)],
            bufs[j % 2], sems[j % 2])

    cp = fire(0)
    for j in range(_CPT):
        off_j = chunk_off(j)
        buf = bufs[j % 2]
        cp.wait()
        if j + 1 < _CPT:
            cp = fire(j + 1)

        def body(vb, carry, j=j, buf=buf, off_j=off_j):
            iv = idx_v[pl.ds(vb * _L, _L)]
            loc = iv - off_j
            valid = jnp.logical_and(loc >= 0, loc < _W)
            locc = jnp.where(valid, loc, 0)
            for f in range(_GF):
                vals = plsc.load_gather(
                    buf, [jnp.full((_L,), f, jnp.int32), locc])
                if j == 0:
                    new = jnp.where(valid, vals, 0.0)
                else:
                    new = jnp.where(valid, vals, outp_v[f, pl.ds(vb * _L, _L)])
                outp_v[f, pl.ds(vb * _L, _L)] = new
            return carry

        lax.fori_loop(0, BATCH // _L, body, 0)

    pltpu.sync_copy(outp_v, shared.at[s])
    plsc.subcore_barrier()

    @pl.when(cpos == 0)
    def _merge_and_write():
        for q in range(1, 4):
            pltpu.sync_copy(shared.at[s + q], tmp_v)

            def mbody(vb, carry):
                for f in range(_GF):
                    sl = pl.ds(vb * _L, _L)
                    outp_v[f, sl] = outp_v[f, sl] + tmp_v[f, sl]
                return carry

            lax.fori_loop(0, BATCH // _L, mbody, 0)
        pltpu.sync_copy(outp_v, outt_hbm.at[pl.ds(f0, _GF), :])


_gather = functools.partial(
    pl.kernel,
    mesh=plsc.VectorSubcoreMesh(core_axis_name="c", subcore_axis_name="s"),
    out_type=jax.ShapeDtypeStruct((DIM, BATCH), jnp.float32),
    scratch_types=[
        pltpu.VMEM((BATCH,), jnp.int32),
        pltpu.VMEM((_GF, _W), jnp.float32),
        pltpu.VMEM((_GF, _W), jnp.float32),
        pltpu.VMEM((_GF, BATCH), jnp.float32),
        pltpu.VMEM((_GF, BATCH), jnp.float32),
        pltpu.VMEM_SHARED((_NS, _GF, BATCH), jnp.float32),
        pltpu.SemaphoreType.DMA,
        pltpu.SemaphoreType.DMA,
    ],
    compiler_params=pltpu.CompilerParams(needs_layout_passes=False),
)(_gather_body)


def kernel(mem, target, idx):
    outt = _gather(mem.T, idx.astype(jnp.int32))
    return (outt.T, target)


# speedup vs baseline: 16.4294x; 1.0741x over previous
import functools

import jax
import jax.numpy as jnp
from jax import lax
from jax.experimental import pallas as pl
from jax.experimental.pallas import tpu as pltpu
from jax.experimental.pallas import tpu_sc as plsc

DS_SIZE = 100000
DIM = 64
BATCH = 1024

_info = plsc.get_sparse_core_info()
_NC, _NS, _L = _info.num_cores, _info.num_subcores, _info.num_lanes
_NGRP = 8
_GF = DIM // _NGRP
_CPT = 4
_W = 6272
_PAD_COLS = 100096
_TAIL_OFF = _PAD_COLS - _W


def _gather_body(memt_hbm, idx_hbm, outt_hbm, idx_v, buf_a, buf_b, outp_v,
                 tmp_v, shared, sem_a, sem_b):
    c = lax.axis_index("c")
    s = lax.axis_index("s")
    grp = c * 4 + s // 4
    cpos = s % 4
    f0 = pl.multiple_of(grp * _GF, _GF)

    pltpu.sync_copy(idx_hbm, idx_v)

    bufs = (buf_a, buf_b)
    sems = (sem_a, sem_b)

    def chunk_off(j):
        off = jnp.minimum((cpos * _CPT + j) * _W, _TAIL_OFF)
        return pl.multiple_of(off, 128)

    def fire(j):
        return pltpu.async_copy(
            memt_hbm.at[pl.ds(f0, _GF), pl.ds(chunk_off(j), _W)],
            bufs[j % 2], sems[j % 2])

    cp = fire(0)
    for j in range(_CPT):
        off_j = chunk_off(j)
        buf = bufs[j % 2]
        cp.wait()
        if j + 1 < _CPT:
            cp = fire(j + 1)

        def body(vb, carry, j=j, buf=buf, off_j=off_j):
            iv = idx_v[pl.ds(vb * _L, _L)]
            loc = iv - off_j
            valid = jnp.logical_and(loc >= 0, loc < _W)
            locc = jnp.where(valid, loc, 0)
            bcols = vb * _L + lax.iota(jnp.int32, _L)
            for f in range(_GF):
                vals = plsc.load_gather(
                    buf, [jnp.full((_L,), f, jnp.int32), locc])
                if j == 0:
                    outp_v[f, pl.ds(vb * _L, _L)] = jnp.where(valid, vals, 0.0)
                else:
                    plsc.store_scatter(
                        outp_v, [jnp.full((_L,), f, jnp.int32), bcols],
                        vals, mask=valid)
            return carry

        lax.fori_loop(0, BATCH // _L, body, 0)

    pltpu.sync_copy(outp_v, shared.at[s])
    plsc.subcore_barrier()

    @pl.when(cpos == 0)
    def _merge_and_write():
        t1 = buf_a.at[:, pl.ds(0, BATCH)]
        t2 = buf_b.at[:, pl.ds(0, BATCH)]
        cp1 = pltpu.async_copy(shared.at[s + 1], t1, sem_a)
        cp2 = pltpu.async_copy(shared.at[s + 2], t2, sem_b)
        cp3 = pltpu.async_copy(shared.at[s + 3], tmp_v, sem_a)
        cp1.wait()
        cp2.wait()
        cp3.wait()

        def mbody(vb, carry):
            for f in range(_GF):
                sl = pl.ds(vb * _L, _L)
                outp_v[f, sl] = ((outp_v[f, sl] + buf_a[f, sl])
                                 + (buf_b[f, sl] + tmp_v[f, sl]))
            return carry

        lax.fori_loop(0, BATCH // _L, mbody, 0)
        pltpu.sync_copy(outp_v, outt_hbm.at[pl.ds(f0, _GF), :])


_gather = functools.partial(
    pl.kernel,
    mesh=plsc.VectorSubcoreMesh(core_axis_name="c", subcore_axis_name="s"),
    out_type=jax.ShapeDtypeStruct((DIM, BATCH), jnp.float32),
    scratch_types=[
        pltpu.VMEM((BATCH,), jnp.int32),
        pltpu.VMEM((_GF, _W), jnp.float32),
        pltpu.VMEM((_GF, _W), jnp.float32),
        pltpu.VMEM((_GF, BATCH), jnp.float32),
        pltpu.VMEM((_GF, BATCH), jnp.float32),
        pltpu.VMEM_SHARED((_NS, _GF, BATCH), jnp.float32),
        pltpu.SemaphoreType.DMA,
        pltpu.SemaphoreType.DMA,
    ],
    compiler_params=pltpu.CompilerParams(needs_layout_passes=False),
)(_gather_body)


def kernel(mem, target, idx):
    outt = _gather(mem.T, idx.astype(jnp.int32))
    return (outt.T, target)
